# baseline (speedup 1.0000x reference)
import functools

import jax
import jax.numpy as jnp
from jax import lax
from jax.experimental import pallas as pl
from jax.experimental.pallas import tpu as pltpu
from jax.experimental.pallas import tpu_sc as plsc

_B = 64
_N = 2048
_S = 16
_D = 256
_K = 32
_STEPS = _S + 1
_HID = 32

_NC = 2
_NS = 16
_BATCH_PER_WORKER = _B // (_NC * _NS)

_CH = 1024
_CPB = _N // _CH
_TOT = _B * _CPB
_NBUF = 16
_NSTG = 4


@functools.cache
def _build_sc_kernel():
    mesh = plsc.VectorSubcoreMesh(
        core_axis_name="c", subcore_axis_name="s", num_cores=_NC, num_subcores=_NS
    )

    @functools.partial(
        pl.kernel,
        out_type=[
            jax.ShapeDtypeStruct((_B, _D), jnp.float32),
            jax.ShapeDtypeStruct((_B, _S), jnp.int32),
        ],
        mesh=mesh,
        compiler_params=pltpu.CompilerParams(needs_layout_passes=False),
        scratch_types=[
            pltpu.VMEM((_S,), jnp.int32),
            pltpu.VMEM((_S, _D), jnp.float32),
            pltpu.VMEM((_D,), jnp.float32),
            pltpu.SemaphoreType.DMA,
        ],
    )
    def sc_body(ent_hbm, sel_hbm, mean_out, sorted_out, idx_v, rows_v, vec_v, sem):
        wid = lax.axis_index("s") * _NC + lax.axis_index("c")
        for j in range(_BATCH_PER_WORKER):
            b = wid * _BATCH_PER_WORKER + j
            pltpu.sync_copy(sel_hbm.at[b], idx_v)
            sv = idx_v[...]
            gidx = sv + b * _N
            pltpu.async_copy(ent_hbm.at[gidx], rows_v, sem).wait()
            for cc in range(_D // 16):
                acc = rows_v[0, pl.ds(cc * 16, 16)]
                for r in range(1, _S):
                    acc = acc + rows_v[r, pl.ds(cc * 16, 16)]
                vec_v[pl.ds(cc * 16, 16)] = acc * (1.0 / _S)
            pltpu.sync_copy(vec_v, mean_out.at[b])
            idx_v[...] = jnp.sort(sv)
            pltpu.sync_copy(idx_v, sorted_out.at[b])

    return sc_body


def _sc_gather_sort(ent_flat, sel):
    return _build_sc_kernel()(ent_flat, sel)


def _ln_k(v, g, bcast_b):
    m = jnp.mean(v, axis=-1, keepdims=True)
    var = jnp.mean((v - m) ** 2, axis=-1, keepdims=True)
    return (v - m) / jnp.sqrt(var + 1e-5) * g + bcast_b


def _mono_body(emb_ref, autm_ref, Wf_ref, bf_ref, W1_ref, b1_ref, W2_ref, b2_ref,
               Wxh_ref, lb_ref, gx_ref, bx_ref, gh_ref, bh_ref, Wk_ref, bk_ref,
               mask_ref, ent_hbm, out_hbm, qv, bv, ebuf, stg, insem, outsem):
    f32 = jnp.float32

    def chunk_src(k):
        b, c = divmod(k, _CPB)
        return ent_hbm.at[b, pl.ds(c * _CH, _CH), :]

    def out_copy(k, slot):
        b, c = divmod(k, _CPB)
        if c < _CPB - 1:
            return pltpu.make_async_copy(
                stg.at[slot, :, pl.ds(0, _CH)],
                out_hbm.at[b, :, pl.ds(c * _CH, _CH)],
                outsem.at[slot])
        return pltpu.make_async_copy(
            stg.at[slot],
            out_hbm.at[b, :, pl.ds(c * _CH, _CH + 1)],
            outsem.at[slot])

    for k in range(_NBUF):
        pltpu.make_async_copy(chunk_src(k), ebuf.at[k], insem.at[k]).start()

    fe = jnp.maximum(
        jnp.dot(autm_ref[...], Wf_ref[...], preferred_element_type=f32) + bf_ref[...], 0.0)
    x0 = jnp.maximum(
        jnp.dot(emb_ref[...], W1_ref[...], preferred_element_type=f32) + b1_ref[...], 0.0)
    x = jnp.maximum(
        jnp.dot(x0 + fe, W2_ref[...], preferred_element_type=f32) + b2_ref[...], 0.0)
    Wxh = Wxh_ref[...]
    lb = lb_ref[...]
    gx = gx_ref[...]
    bx = bx_ref[...]
    gh = gh_ref[...]
    bh = bh_ref[...]
    Wk = Wk_ref[...]
    bk = bk_ref[...]
    c_st = jnp.zeros((_B, _HID), f32)
    h = jnp.zeros((_B, _HID), f32)
    for s in range(_STEPS):
        if s == 0:
            zx = jnp.dot(x, Wxh_ref[:, 0:4 * _HID], preferred_element_type=f32)
            gates = _ln_k(zx, gx, bx) + bh + lb
        else:
            z = jnp.dot(h, Wxh, preferred_element_type=f32)
            gates = (_ln_k(z[:, 0:4 * _HID], gx, bx)
                     + _ln_k(z[:, 4 * _HID:8 * _HID], gh, bh) + lb)
        sa = jax.nn.sigmoid(gates)
        tg = jnp.tanh(gates[:, 2 * _HID:3 * _HID])
        c_st = sa[:, _HID:2 * _HID] * c_st + sa[:, 0:_HID] * tg
        th = jnp.tanh(c_st)
        h = sa[:, 3 * _HID:4 * _HID] * th
        qv[:, s, :] = lax.dot_general(h, Wk, (((1,), (1,)), ((), ())),
                                      preferred_element_type=f32)
        bv[:, s, :] = jnp.sum(h * bk, axis=1, keepdims=True)

    for k in range(_TOT):
        slot = k % _NBUF
        b, c = divmod(k, _CPB)
        pltpu.make_async_copy(chunk_src(k), ebuf.at[slot], insem.at[slot]).wait()
        q = qv[b]
        e = ebuf[slot]
        mm = lax.dot_general(q, e, (((1,), (1,)), ((), ())),
                             preferred_element_type=f32)
        bias = bv[b]
        m = mask_ref[pl.ds(b, 1), pl.ds(c * _CH, _CH)]
        res = (mm + bias) * (1.0 / _K) - (1.0 - m) * 1e9
        s2 = k % _NSTG
        if k >= _NSTG:
            out_copy(k - _NSTG, s2).wait()
        stg[s2, :, 0:_CH] = res
        if c == _CPB - 1:
            stg[s2, :, _CH:_CH + 1] = jnp.zeros((_STEPS, 1), f32)
        out_copy(k, s2).start()
        if k + _NBUF < _TOT:
            pltpu.make_async_copy(chunk_src(k + _NBUF), ebuf.at[slot],
                                  insem.at[slot]).start()

    for k in range(max(0, _TOT - _NSTG), _TOT):
        out_copy(k, k % _NSTG).wait()


def _tail_body(mean_ref, Wk_ref, bk_ref, We_ref, be_ref, emb_ref, out_ref):
    f32 = jnp.float32
    ksel = jnp.dot(mean_ref[...], Wk_ref[...], preferred_element_type=f32) + bk_ref[...]
    out_ref[...] = (emb_ref[...]
                    + jnp.dot(ksel, We_ref[...], preferred_element_type=f32) + be_ref[...])


def kernel(embedding, available_unit_type_mask, available_units_mask, entity_embedding,
           selected_units, key_fc_W, key_fc_b, func_fc_W, func_fc_b, fc1_W, fc1_b,
           fc2_W, fc2_b, embed_fc_W, embed_fc_b, lstm_Wx, lstm_Wh, lstm_b,
           ln_gx, ln_bx, ln_gh, ln_bh):
    f32 = jnp.float32
    r2 = lambda a: a.reshape(1, -1)

    Wxh = jnp.concatenate([lstm_Wx, lstm_Wh], axis=1)

    logits = pl.pallas_call(
        _mono_body,
        in_specs=[pl.BlockSpec(memory_space=pl.ANY) if i == 17
                  else pl.BlockSpec() for i in range(18)],
        out_specs=pl.BlockSpec(memory_space=pl.ANY),
        out_shape=jax.ShapeDtypeStruct((_B, _STEPS, _N + 1), f32),
        scratch_shapes=[
            pltpu.VMEM((_B, _STEPS, _D), f32),
            pltpu.VMEM((_B, _STEPS, 1), f32),
            pltpu.VMEM((_NBUF, _CH, _D), f32),
            pltpu.VMEM((_NSTG, _STEPS, _CH + 1), f32),
            pltpu.SemaphoreType.DMA((_NBUF,)),
            pltpu.SemaphoreType.DMA((_NSTG,)),
        ],
    )(embedding, available_unit_type_mask, func_fc_W, r2(func_fc_b),
      fc1_W, r2(fc1_b), fc2_W, r2(fc2_b),
      Wxh, r2(lstm_b), r2(ln_gx), r2(ln_bx), r2(ln_gh), r2(ln_bh),
      key_fc_W, r2(key_fc_b), available_units_mask, entity_embedding)

    ent_flat = entity_embedding.reshape(_B * _N, _D)
    mean_sel, units_index = _sc_gather_sort(ent_flat, selected_units)

    out3 = pl.pallas_call(
        _tail_body,
        out_shape=jax.ShapeDtypeStruct((_B, 1024), f32),
    )(mean_sel, key_fc_W, r2(key_fc_b), embed_fc_W, r2(embed_fc_b), embedding)

    return (logits, units_index, out3)


# speedup vs baseline: 1.0003x; 1.0003x over previous
import functools

import jax
import jax.numpy as jnp
from jax import lax
from jax.experimental import pallas as pl
from jax.experimental.pallas import tpu as pltpu
from jax.experimental.pallas import tpu_sc as plsc

_B = 64
_N = 2048
_S = 16
_D = 256
_K = 32
_STEPS = _S + 1
_HID = 32

_NC = 2
_NS = 16
_BATCH_PER_WORKER = _B // (_NC * _NS)

_CH = 1024
_CPB = _N // _CH
_TOT = _B * _CPB
_NBUF = 16
_NSTG = 4


@functools.cache
def _build_sc_kernel():
    mesh = plsc.VectorSubcoreMesh(
        core_axis_name="c", subcore_axis_name="s", num_cores=_NC, num_subcores=_NS
    )

    @functools.partial(
        pl.kernel,
        out_type=[
            jax.ShapeDtypeStruct((_B, _D), jnp.float32),
            jax.ShapeDtypeStruct((_B, _S), jnp.int32),
        ],
        mesh=mesh,
        compiler_params=pltpu.CompilerParams(needs_layout_passes=False),
        scratch_types=[
            pltpu.VMEM((_BATCH_PER_WORKER, _S), jnp.int32),
            pltpu.VMEM((_BATCH_PER_WORKER * _S,), jnp.int32),
            pltpu.VMEM((_BATCH_PER_WORKER * _S, _D), jnp.float32),
            pltpu.VMEM((_BATCH_PER_WORKER, _D), jnp.float32),
            pltpu.VMEM((_BATCH_PER_WORKER, _S), jnp.int32),
            pltpu.SemaphoreType.DMA,
            pltpu.SemaphoreType.DMA,
            pltpu.SemaphoreType.DMA,
        ],
    )
    def sc_body(ent_hbm, sel_hbm, mean_out, sorted_out,
                sel_v, gidx_v, rows_v, mean_v, srt_v, sem, sem_m, sem_s):
        wid = lax.axis_index("s") * _NC + lax.axis_index("c")
        b0 = wid * _BATCH_PER_WORKER
        pltpu.sync_copy(sel_hbm.at[pl.ds(b0, _BATCH_PER_WORKER)], sel_v)
        svs = []
        for j in range(_BATCH_PER_WORKER):
            sv = sel_v[j]
            svs.append(sv)
            gidx_v[pl.ds(j * _S, _S)] = sv + (b0 + j) * _N
        pltpu.async_copy(ent_hbm.at[gidx_v], rows_v, sem).wait()
        for j in range(_BATCH_PER_WORKER):
            for cc in range(_D // 16):
                acc = rows_v[j * _S, pl.ds(cc * 16, 16)]
                for r in range(1, _S):
                    acc = acc + rows_v[j * _S + r, pl.ds(cc * 16, 16)]
                mean_v[j, pl.ds(cc * 16, 16)] = acc * (1.0 / _S)
            srt_v[j] = jnp.sort(svs[j])
        cp_m = pltpu.async_copy(mean_v, mean_out.at[pl.ds(b0, _BATCH_PER_WORKER)], sem_m)
        cp_s = pltpu.async_copy(srt_v, sorted_out.at[pl.ds(b0, _BATCH_PER_WORKER)], sem_s)
        cp_m.wait()
        cp_s.wait()

    return sc_body


def _sc_gather_sort(ent_flat, sel):
    return _build_sc_kernel()(ent_flat, sel)


def _ln_k(v, g, bcast_b):
    m = jnp.mean(v, axis=-1, keepdims=True)
    var = jnp.mean((v - m) ** 2, axis=-1, keepdims=True)
    return (v - m) / jnp.sqrt(var + 1e-5) * g + bcast_b


def _mono_body(emb_ref, autm_ref, Wf_ref, bf_ref, W1_ref, b1_ref, W2_ref, b2_ref,
               Wxh_ref, lb_ref, gx_ref, bx_ref, gh_ref, bh_ref, Wk_ref, bk_ref,
               mask_ref, ent_hbm, out_hbm, qv, bv, ebuf, stg, insem, outsem):
    f32 = jnp.float32

    def chunk_src(k):
        b, c = divmod(k, _CPB)
        return ent_hbm.at[b, pl.ds(c * _CH, _CH), :]

    def out_copy(k, slot):
        b, c = divmod(k, _CPB)
        if c < _CPB - 1:
            return pltpu.make_async_copy(
                stg.at[slot, :, pl.ds(0, _CH)],
                out_hbm.at[b, :, pl.ds(c * _CH, _CH)],
                outsem.at[slot])
        return pltpu.make_async_copy(
            stg.at[slot],
            out_hbm.at[b, :, pl.ds(c * _CH, _CH + 1)],
            outsem.at[slot])

    for k in range(_NBUF):
        pltpu.make_async_copy(chunk_src(k), ebuf.at[k], insem.at[k]).start()

    fe = jnp.maximum(
        jnp.dot(autm_ref[...], Wf_ref[...], preferred_element_type=f32) + bf_ref[...], 0.0)
    x0 = jnp.maximum(
        jnp.dot(emb_ref[...], W1_ref[...], preferred_element_type=f32) + b1_ref[...], 0.0)
    x = jnp.maximum(
        jnp.dot(x0 + fe, W2_ref[...], preferred_element_type=f32) + b2_ref[...], 0.0)
    Wxh = Wxh_ref[...]
    lb = lb_ref[...]
    gx = gx_ref[...]
    bx = bx_ref[...]
    gh = gh_ref[...]
    bh = bh_ref[...]
    Wk = Wk_ref[...]
    bk = bk_ref[...]
    c_st = jnp.zeros((_B, _HID), f32)
    h = jnp.zeros((_B, _HID), f32)
    for s in range(_STEPS):
        if s == 0:
            zx = jnp.dot(x, Wxh_ref[:, 0:4 * _HID], preferred_element_type=f32)
            gates = _ln_k(zx, gx, bx) + bh + lb
        else:
            z = jnp.dot(h, Wxh, preferred_element_type=f32)
            gates = (_ln_k(z[:, 0:4 * _HID], gx, bx)
                     + _ln_k(z[:, 4 * _HID:8 * _HID], gh, bh) + lb)
        sa = jax.nn.sigmoid(gates)
        tg = jnp.tanh(gates[:, 2 * _HID:3 * _HID])
        c_st = sa[:, _HID:2 * _HID] * c_st + sa[:, 0:_HID] * tg
        th = jnp.tanh(c_st)
        h = sa[:, 3 * _HID:4 * _HID] * th
        qv[:, s, :] = lax.dot_general(h, Wk, (((1,), (1,)), ((), ())),
                                      preferred_element_type=f32)
        bv[:, s, :] = jnp.sum(h * bk, axis=1, keepdims=True)

    for k in range(_TOT):
        slot = k % _NBUF
        b, c = divmod(k, _CPB)
        pltpu.make_async_copy(chunk_src(k), ebuf.at[slot], insem.at[slot]).wait()
        q = qv[b]
        e = ebuf[slot]
        mm = lax.dot_general(q, e, (((1,), (1,)), ((), ())),
                             preferred_element_type=f32)
        bias = bv[b]
        m = mask_ref[pl.ds(b, 1), pl.ds(c * _CH, _CH)]
        res = (mm + bias) * (1.0 / _K) - (1.0 - m) * 1e9
        s2 = k % _NSTG
        if k >= _NSTG:
            out_copy(k - _NSTG, s2).wait()
        stg[s2, :, 0:_CH] = res
        if c == _CPB - 1:
            stg[s2, :, _CH:_CH + 1] = jnp.zeros((_STEPS, 1), f32)
        out_copy(k, s2).start()
        if k + _NBUF < _TOT:
            pltpu.make_async_copy(chunk_src(k + _NBUF), ebuf.at[slot],
                                  insem.at[slot]).start()

    for k in range(max(0, _TOT - _NSTG), _TOT):
        out_copy(k, k % _NSTG).wait()


def _tail_body(mean_ref, Wk_ref, bk_ref, We_ref, be_ref, emb_ref, out_ref):
    f32 = jnp.float32
    ksel = jnp.dot(mean_ref[...], Wk_ref[...], preferred_element_type=f32) + bk_ref[...]
    out_ref[...] = (emb_ref[...]
                    + jnp.dot(ksel, We_ref[...], preferred_element_type=f32) + be_ref[...])


def kernel(embedding, available_unit_type_mask, available_units_mask, entity_embedding,
           selected_units, key_fc_W, key_fc_b, func_fc_W, func_fc_b, fc1_W, fc1_b,
           fc2_W, fc2_b, embed_fc_W, embed_fc_b, lstm_Wx, lstm_Wh, lstm_b,
           ln_gx, ln_bx, ln_gh, ln_bh):
    f32 = jnp.float32
    r2 = lambda a: a.reshape(1, -1)

    ent_flat = entity_embedding.reshape(_B * _N, _D)
    mean_sel, units_index = _sc_gather_sort(ent_flat, selected_units)

    Wxh = jnp.concatenate([lstm_Wx, lstm_Wh], axis=1)

    logits = pl.pallas_call(
        _mono_body,
        in_specs=[pl.BlockSpec(memory_space=pl.ANY) if i == 17
                  else pl.BlockSpec() for i in range(18)],
        out_specs=pl.BlockSpec(memory_space=pl.ANY),
        out_shape=jax.ShapeDtypeStruct((_B, _STEPS, _N + 1), f32),
        scratch_shapes=[
            pltpu.VMEM((_B, _STEPS, _D), f32),
            pltpu.VMEM((_B, _STEPS, 1), f32),
            pltpu.VMEM((_NBUF, _CH, _D), f32),
            pltpu.VMEM((_NSTG, _STEPS, _CH + 1), f32),
            pltpu.SemaphoreType.DMA((_NBUF,)),
            pltpu.SemaphoreType.DMA((_NSTG,)),
        ],
    )(embedding, available_unit_type_mask, func_fc_W, r2(func_fc_b),
      fc1_W, r2(fc1_b), fc2_W, r2(fc2_b),
      Wxh, r2(lstm_b), r2(ln_gx), r2(ln_bx), r2(ln_gh), r2(ln_bh),
      key_fc_W, r2(key_fc_b), available_units_mask, entity_embedding)

    out3 = pl.pallas_call(
        _tail_body,
        out_shape=jax.ShapeDtypeStruct((_B, 1024), f32),
    )(mean_sel, key_fc_W, r2(key_fc_b), embed_fc_W, r2(embed_fc_b), embedding)

    return (logits, units_index, out3)


# speedup vs baseline: 1.1972x; 1.1968x over previous
import functools

import jax
import jax.numpy as jnp
from jax import lax
from jax.experimental import pallas as pl
from jax.experimental.pallas import tpu as pltpu
from jax.experimental.pallas import tpu_sc as plsc

_B = 64
_N = 2048
_S = 16
_D = 256
_K = 32
_STEPS = _S + 1
_HID = 32

_NC = 2
_NS = 16
_BATCH_PER_WORKER = _B // (_NC * _NS)

_CH = 1024
_CPB = _N // _CH
_TOT = _B * _CPB
_NBUF = 16
_NSTG = 4


@functools.cache
def _build_sc_kernel():
    mesh = plsc.VectorSubcoreMesh(
        core_axis_name="c", subcore_axis_name="s", num_cores=_NC, num_subcores=_NS
    )

    @functools.partial(
        pl.kernel,
        out_type=[
            jax.ShapeDtypeStruct((_B, _D), jnp.float32),
            jax.ShapeDtypeStruct((_B, _S), jnp.int32),
        ],
        mesh=mesh,
        compiler_params=pltpu.CompilerParams(needs_layout_passes=False),
        scratch_types=[
            pltpu.VMEM((_S, _B), jnp.int32),
            pltpu.VMEM((_BATCH_PER_WORKER * _S,), jnp.int32),
            pltpu.VMEM((_BATCH_PER_WORKER * _S, _D), jnp.float32),
            pltpu.VMEM((_BATCH_PER_WORKER, _D), jnp.float32),
            pltpu.VMEM((_BATCH_PER_WORKER, _S), jnp.int32),
            pltpu.SemaphoreType.DMA,
            pltpu.SemaphoreType.DMA,
            pltpu.SemaphoreType.DMA,
        ],
    )
    def sc_body(ent_hbm, selT_hbm, mean_out, sorted_out,
                sel_v, gidx_v, rows_v, mean_v, srt_v, sem, sem_m, sem_s):
        wid = lax.axis_index("s") * _NC + lax.axis_index("c")
        b0 = wid * _BATCH_PER_WORKER
        pltpu.sync_copy(selT_hbm, sel_v)
        lane = lax.iota(jnp.int32, _S)
        svs = []
        for j in range(_BATCH_PER_WORKER):
            sv = plsc.load_gather(sel_v, [lane, jnp.full((_S,), b0 + j, jnp.int32)])
            svs.append(sv)
            gidx_v[pl.ds(j * _S, _S)] = sv + (b0 + j) * _N
        pltpu.async_copy(ent_hbm.at[gidx_v], rows_v, sem).wait()
        for j in range(_BATCH_PER_WORKER):
            for cc in range(_D // 16):
                acc = rows_v[j * _S, pl.ds(cc * 16, 16)]
                for r in range(1, _S):
                    acc = acc + rows_v[j * _S + r, pl.ds(cc * 16, 16)]
                mean_v[j, pl.ds(cc * 16, 16)] = acc * (1.0 / _S)
            srt_v[j] = jnp.sort(svs[j])
        cp_m = pltpu.async_copy(mean_v, mean_out.at[pl.ds(b0, _BATCH_PER_WORKER)], sem_m)
        cp_s = pltpu.async_copy(srt_v, sorted_out.at[pl.ds(b0, _BATCH_PER_WORKER)], sem_s)
        cp_m.wait()
        cp_s.wait()

    return sc_body


def _sc_gather_sort(ent_flat, sel):
    return _build_sc_kernel()(ent_flat, sel)


def _ln_k(v, g, bcast_b):
    m = jnp.mean(v, axis=-1, keepdims=True)
    var = jnp.mean((v - m) ** 2, axis=-1, keepdims=True)
    return (v - m) / jnp.sqrt(var + 1e-5) * g + bcast_b


def _mono_body(emb_ref, autm_ref, Wf_ref, bf_ref, W1_ref, b1_ref, W2T_ref, b2_ref,
               Wx_ref, Wh_ref, lb_ref, gx_ref, bx_ref, gh_ref, bh_ref, WkT_ref, bk_ref,
               mask_ref, ent_hbm, out_hbm, qv, bv, ebuf, stg, insem, outsem):
    f32 = jnp.float32

    def chunk_src(k):
        b, c = divmod(k, _CPB)
        return ent_hbm.at[b, pl.ds(c * _CH, _CH), :]

    def out_copy(k, slot):
        b, c = divmod(k, _CPB)
        if c < _CPB - 1:
            return pltpu.make_async_copy(
                stg.at[slot, :, pl.ds(0, _CH)],
                out_hbm.at[:, b, pl.ds(c * _CH, _CH)],
                outsem.at[slot])
        return pltpu.make_async_copy(
            stg.at[slot],
            out_hbm.at[:, b, pl.ds(c * _CH, _CH + 1)],
            outsem.at[slot])

    for k in range(_NBUF):
        pltpu.make_async_copy(chunk_src(k), ebuf.at[k], insem.at[k]).start()

    fe = jnp.maximum(
        jnp.dot(autm_ref[...], Wf_ref[...], preferred_element_type=f32) + bf_ref[...], 0.0)
    x0 = jnp.maximum(
        jnp.dot(emb_ref[...], W1_ref[...], preferred_element_type=f32) + b1_ref[...], 0.0)
    x = jnp.maximum(
        lax.dot_general(x0 + fe, W2T_ref[...], (((1,), (1,)), ((), ())),
                        preferred_element_type=f32) + b2_ref[...], 0.0)
    Wxh = jnp.concatenate([Wx_ref[...], Wh_ref[...]], axis=1)
    lb = lb_ref[...]
    gx = gx_ref[...]
    bx = bx_ref[...]
    gh = gh_ref[...]
    bh = bh_ref[...]
    WkT = WkT_ref[...]
    bk = bk_ref[...]
    c_st = jnp.zeros((_B, _HID), f32)
    h = jnp.zeros((_B, _HID), f32)
    for s in range(_STEPS):
        if s == 0:
            zx = jnp.dot(x, Wx_ref[...], preferred_element_type=f32)
            gates = _ln_k(zx, gx, bx) + bh + lb
        else:
            z = jnp.dot(h, Wxh, preferred_element_type=f32)
            gates = (_ln_k(z[:, 0:4 * _HID], gx, bx)
                     + _ln_k(z[:, 4 * _HID:8 * _HID], gh, bh) + lb)
        sa = jax.nn.sigmoid(gates)
        tg = jnp.tanh(gates[:, 2 * _HID:3 * _HID])
        c_st = sa[:, _HID:2 * _HID] * c_st + sa[:, 0:_HID] * tg
        th = jnp.tanh(c_st)
        h = sa[:, 3 * _HID:4 * _HID] * th
        qv[:, s, :] = lax.dot_general(h, WkT, (((1,), (0,)), ((), ())),
                                      preferred_element_type=f32)
        bv[:, s, :] = jnp.sum(h * bk, axis=1, keepdims=True)

    for k in range(_TOT):
        slot = k % _NBUF
        b, c = divmod(k, _CPB)
        pltpu.make_async_copy(chunk_src(k), ebuf.at[slot], insem.at[slot]).wait()
        q = qv[b]
        e = ebuf[slot]
        mm = lax.dot_general(q, e, (((1,), (1,)), ((), ())),
                             preferred_element_type=f32)
        bias = bv[b]
        m = mask_ref[pl.ds(b, 1), pl.ds(c * _CH, _CH)]
        res = (mm + bias) * (1.0 / _K) - (1.0 - m) * 1e9
        s2 = k % _NSTG
        if k >= _NSTG:
            out_copy(k - _NSTG, s2).wait()
        stg[s2, :, 0:_CH] = res
        if c == _CPB - 1:
            stg[s2, :, _CH:_CH + 1] = jnp.zeros((_STEPS, 1), f32)
        out_copy(k, s2).start()
        if k + _NBUF < _TOT:
            pltpu.make_async_copy(chunk_src(k + _NBUF), ebuf.at[slot],
                                  insem.at[slot]).start()

    for k in range(max(0, _TOT - _NSTG), _TOT):
        out_copy(k, k % _NSTG).wait()


def _tail_body(mean_ref, WkT_ref, bk_ref, We_ref, be_ref, emb_ref, sorted_ref,
               out_ref, unitsT_ref):
    f32 = jnp.float32
    ksel = lax.dot_general(mean_ref[...], WkT_ref[...], (((1,), (1,)), ((), ())),
                           preferred_element_type=f32) + bk_ref[...]
    out_ref[...] = (emb_ref[...]
                    + jnp.dot(ksel, We_ref[...], preferred_element_type=f32) + be_ref[...])
    unitsT_ref[...] = sorted_ref[...].T


def kernel(embedding, available_unit_type_mask, available_units_mask, entity_embedding,
           selected_units, key_fc_W, key_fc_b, func_fc_W, func_fc_b, fc1_W, fc1_b,
           fc2_W, fc2_b, embed_fc_W, embed_fc_b, lstm_Wx, lstm_Wh, lstm_b,
           ln_gx, ln_bx, ln_gh, ln_bh):
    f32 = jnp.float32
    r2 = lambda a: a.reshape(1, -1)

    ent_flat = entity_embedding.reshape(_B * _N, _D)
    mean_sel, units_sorted = _sc_gather_sort(ent_flat, selected_units.T)

    W2T = fc2_W.T
    WkT = key_fc_W.T

    logitsT = pl.pallas_call(
        _mono_body,
        in_specs=[pl.BlockSpec(memory_space=pl.ANY) if i == 18
                  else pl.BlockSpec() for i in range(19)],
        out_specs=pl.BlockSpec(memory_space=pl.ANY),
        out_shape=jax.ShapeDtypeStruct((_STEPS, _B, _N + 1), f32),
        scratch_shapes=[
            pltpu.VMEM((_B, _STEPS, _D), f32),
            pltpu.VMEM((_B, _STEPS, 1), f32),
            pltpu.VMEM((_NBUF, _CH, _D), f32),
            pltpu.VMEM((_NSTG, _STEPS, _CH + 1), f32),
            pltpu.SemaphoreType.DMA((_NBUF,)),
            pltpu.SemaphoreType.DMA((_NSTG,)),
        ],
    )(embedding, available_unit_type_mask, func_fc_W, r2(func_fc_b),
      fc1_W, r2(fc1_b), W2T, r2(fc2_b),
      lstm_Wx, lstm_Wh, r2(lstm_b), r2(ln_gx), r2(ln_bx), r2(ln_gh), r2(ln_bh),
      WkT, r2(key_fc_b), available_units_mask, entity_embedding)
    logits = logitsT.transpose(1, 0, 2)

    out3, unitsT = pl.pallas_call(
        _tail_body,
        out_shape=[
            jax.ShapeDtypeStruct((_B, 1024), f32),
            jax.ShapeDtypeStruct((_S, _B), jnp.int32),
        ],
    )(mean_sel, WkT, r2(key_fc_b), embed_fc_W, r2(embed_fc_b), embedding,
      units_sorted)
    units_index = unitsT.T

    return (logits, units_index, out3)
